# Initial kernel scaffold; baseline (speedup 1.0000x reference)
import jax
import jax.numpy as jnp
from jax.experimental import pallas as pl


def kernel(features, edge_index, W_sg, b_sg, W_lin, b_lin):
    raise NotImplementedError("write your pallas kernel here")


# speedup vs baseline: 8.7146x; 8.7146x over previous
import functools

import jax
import jax.numpy as jnp
from jax import lax
from jax.experimental import pallas as pl
from jax.experimental.pallas import tpu as pltpu
from jax.experimental.pallas import tpu_sc as plsc

N_NODES = 10000
N_EDGES = 320000
D_FEAT = 128
N_CLASSES = 40

NC = 2
NS = 16
NW = NC * NS
CH = 128
GC = 8
E_PAD = 327680
EPW = E_PAD // NW
NCHUNK = EPW // CH
NG = NCHUNK // GC
N_PAD = 10240
RPT = N_PAD // NS
DEGW = 128

_mesh = plsc.VectorSubcoreMesh(core_axis_name="c", subcore_axis_name="s")


def _zero_vmem(ref, nrows, width):
    z = jnp.zeros((16,), jnp.float32)

    def body(i, carry):
        for k in range(width // 16):
            ref[i, pl.ds(k * 16, 16)] = z
        return carry

    lax.fori_loop(0, nrows, body, 0)


@functools.partial(
    pl.kernel,
    out_type=jax.ShapeDtypeStruct((NC, N_PAD, DEGW), jnp.float32),
    mesh=_mesh,
    scratch_types=[
        pltpu.VMEM((NCHUNK, CH), jnp.int32),
        pltpu.VMEM((CH, DEGW), jnp.float32),
        pltpu.VMEM((CH, DEGW), jnp.float32),
        pltpu.VMEM_SHARED((N_PAD, DEGW), jnp.float32),
    ],
)
def _deg_kernel(dst_hbm, out_hbm, dstv, onesv, stg, acc):
    c = lax.axis_index("c")
    s = lax.axis_index("s")
    w = c * NS + s
    one = jnp.ones((16,), jnp.float32)

    def init_ones(i, carry):
        for k in range(DEGW // 16):
            onesv[i, pl.ds(k * 16, 16)] = one
        return carry

    lax.fori_loop(0, CH, init_ones, 0)
    _zero_vmem(stg, CH, DEGW)
    base = s * RPT
    for b in range(RPT // CH):
        pltpu.sync_copy(stg, acc.at[pl.ds(base + b * CH, CH)])
    plsc.subcore_barrier()

    pltpu.sync_copy(dst_hbm.at[w], dstv)

    def chunk(j, carry):
        pltpu.sync_copy(onesv, acc.at[dstv.at[j]], add=True)
        return carry

    lax.fori_loop(0, NCHUNK, chunk, 0)
    plsc.subcore_barrier()

    for b in range(RPT // CH):
        r0 = base + b * CH
        pltpu.sync_copy(acc.at[pl.ds(r0, CH)], stg)
        pltpu.sync_copy(stg, out_hbm.at[c, pl.ds(r0, CH)])


@functools.partial(
    pl.kernel,
    out_type=jax.ShapeDtypeStruct((NC, N_PAD, D_FEAT), jnp.float32),
    mesh=_mesh,
    scratch_types=[
        pltpu.VMEM((GC, CH), jnp.int32),
        pltpu.VMEM((NCHUNK, CH), jnp.int32),
        pltpu.VMEM((CH, D_FEAT), jnp.float32),
        pltpu.VMEM((CH, D_FEAT), jnp.float32),
        pltpu.VMEM_SHARED((N_PAD, D_FEAT), jnp.float32),
        pltpu.SemaphoreType.DMA,
        pltpu.SemaphoreType.DMA,
    ],
)
def _hop_kernel(g_hbm, src_hbm, dst_hbm, out_hbm, srcg, dstv, rows0, rows1, acc, sem0, sem1):
    c = lax.axis_index("c")
    s = lax.axis_index("s")
    w = c * NS + s

    _zero_vmem(rows0, CH, D_FEAT)
    base = s * RPT
    for b in range(RPT // CH):
        pltpu.sync_copy(rows0, acc.at[pl.ds(base + b * CH, CH)])
    plsc.subcore_barrier()

    pltpu.sync_copy(dst_hbm.at[w], dstv)

    bufs = (rows0, rows1)
    sems = (sem0, sem1)

    def group(g, carry):
        pltpu.sync_copy(src_hbm.at[w, g], srcg)
        pltpu.async_copy(g_hbm.at[srcg.at[0]], rows0, sem0)
        pltpu.async_copy(g_hbm.at[srcg.at[1]], rows1, sem1)
        for k in range(GC):
            buf, sem = bufs[k % 2], sems[k % 2]
            pltpu.make_async_copy(g_hbm.at[srcg.at[k]], buf, sem).wait()
            pltpu.sync_copy(buf, acc.at[dstv.at[g * GC + k]], add=True)
            if k + 2 < GC:
                pltpu.async_copy(g_hbm.at[srcg.at[k + 2]], buf, sem)
        return carry

    lax.fori_loop(0, NG, group, 0)
    plsc.subcore_barrier()

    for b in range(RPT // CH):
        r0 = base + b * CH
        pltpu.sync_copy(acc.at[pl.ds(r0, CH)], rows0)
        pltpu.sync_copy(rows0, out_hbm.at[c, pl.ds(r0, CH)])


_ROWS_BLK = 1000
_GRID = N_NODES // _ROWS_BLK


def _prep_body(f_ref, degp_ref, g0_ref, nrm_ref):
    deg = jnp.maximum(degp_ref[0][:, 0:1] + degp_ref[1][:, 0:1], 1.0)
    rn = lax.rsqrt(deg)
    inv = 1.0 / deg
    g0_ref[...] = f_ref[...] * rn
    nrm_ref[...] = jnp.concatenate([rn, inv, rn, inv, rn, inv, rn, inv], axis=1)


def _mid_body(p_ref, nrm_ref, g1_ref):
    g1_ref[...] = (p_ref[0] + p_ref[1]) * nrm_ref[:, 1:2]


def _head_body(q_ref, nrm_ref, wsg_ref, bsg_ref, wlin_ref, blin_ref, o_ref):
    h = (q_ref[0] + q_ref[1]) * nrm_ref[:, 0:1]
    z = jnp.dot(h, wsg_ref[...], preferred_element_type=jnp.float32) + bsg_ref[...]
    o_ref[...] = (
        jnp.dot(jnp.maximum(z, 0.0), wlin_ref[...], preferred_element_type=jnp.float32)
        + blin_ref[...]
    )


_degp_spec = pl.BlockSpec((NC, _ROWS_BLK, DEGW), lambda i: (0, i, 0))
_full_spec = pl.BlockSpec((_ROWS_BLK, D_FEAT), lambda i: (i, 0))
_pair_spec = pl.BlockSpec((NC, _ROWS_BLK, D_FEAT), lambda i: (0, i, 0))
_nrm_spec = pl.BlockSpec((_ROWS_BLK, 8), lambda i: (i, 0))

_prep = pl.pallas_call(
    _prep_body,
    grid=(_GRID,),
    in_specs=[_full_spec, _degp_spec],
    out_specs=(_full_spec, _nrm_spec),
    out_shape=(
        jax.ShapeDtypeStruct((N_NODES, D_FEAT), jnp.float32),
        jax.ShapeDtypeStruct((N_NODES, 8), jnp.float32),
    ),
)

_mid = pl.pallas_call(
    _mid_body,
    grid=(_GRID,),
    in_specs=[_pair_spec, _nrm_spec],
    out_specs=_full_spec,
    out_shape=jax.ShapeDtypeStruct((N_NODES, D_FEAT), jnp.float32),
)

_head = pl.pallas_call(
    _head_body,
    grid=(_GRID,),
    in_specs=[
        _pair_spec,
        _nrm_spec,
        pl.BlockSpec((D_FEAT, D_FEAT), lambda i: (0, 0)),
        pl.BlockSpec((1, D_FEAT), lambda i: (0, 0)),
        pl.BlockSpec((D_FEAT, N_CLASSES), lambda i: (0, 0)),
        pl.BlockSpec((1, N_CLASSES), lambda i: (0, 0)),
    ],
    out_specs=pl.BlockSpec((_ROWS_BLK, N_CLASSES), lambda i: (i, 0)),
    out_shape=jax.ShapeDtypeStruct((N_NODES, N_CLASSES), jnp.float32),
)


def kernel(features, edge_index, W_sg, b_sg, W_lin, b_lin):
    n_extra = E_PAD - N_EDGES
    pad_ids = jnp.arange(n_extra, dtype=jnp.int32)
    pad_src = pad_ids % N_NODES
    pad_dst = N_NODES + pad_ids % (N_PAD - N_NODES)
    src = jnp.concatenate([edge_index[0], pad_src]).reshape(NW, NG, GC, CH)
    dst = jnp.concatenate([edge_index[1], pad_dst]).reshape(NW, NCHUNK, CH)
    degp = _deg_kernel(dst)
    g0, nrm = _prep(features, degp)
    p1 = _hop_kernel(g0, src, dst)
    g1 = _mid(p1, nrm)
    p2 = _hop_kernel(g1, src, dst)
    return _head(
        p2, nrm, W_sg, b_sg.reshape(1, D_FEAT), W_lin, b_lin.reshape(1, N_CLASSES)
    )


# speedup vs baseline: 8.7310x; 1.0019x over previous
import functools

import jax
import jax.numpy as jnp
from jax import lax
from jax.experimental import pallas as pl
from jax.experimental.pallas import tpu as pltpu
from jax.experimental.pallas import tpu_sc as plsc

N_NODES = 10000
N_EDGES = 320000
D_FEAT = 128
N_CLASSES = 40

NC = 2
NS = 16
NW = NC * NS
CH = 128
GC = 8
E_PAD = 327680
EPW = E_PAD // NW
NCHUNK = EPW // CH
NG = NCHUNK // GC
N_PAD = 10240
RPT = N_PAD // NS
DEGW = 128

_mesh = plsc.VectorSubcoreMesh(core_axis_name="c", subcore_axis_name="s")


def _zero_vmem(ref, nrows, width):
    z = jnp.zeros((16,), jnp.float32)

    def body(i, carry):
        for k in range(width // 16):
            ref[i, pl.ds(k * 16, 16)] = z
        return carry

    lax.fori_loop(0, nrows, body, 0)


@functools.partial(
    pl.kernel,
    out_type=jax.ShapeDtypeStruct((NC, N_PAD, DEGW), jnp.float32),
    mesh=_mesh,
    scratch_types=[
        pltpu.VMEM((NCHUNK, CH), jnp.int32),
        pltpu.VMEM((CH, DEGW), jnp.float32),
        pltpu.VMEM((CH, DEGW), jnp.float32),
        pltpu.VMEM_SHARED((N_PAD, DEGW), jnp.float32),
    ],
)
def _deg_kernel(dst_hbm, out_hbm, dstv, onesv, stg, acc):
    c = lax.axis_index("c")
    s = lax.axis_index("s")
    w = c * NS + s
    one = jnp.ones((16,), jnp.float32)

    def init_ones(i, carry):
        for k in range(DEGW // 16):
            onesv[i, pl.ds(k * 16, 16)] = one
        return carry

    lax.fori_loop(0, CH, init_ones, 0)
    _zero_vmem(stg, CH, DEGW)
    base = s * RPT
    for b in range(RPT // CH):
        pltpu.sync_copy(stg, acc.at[pl.ds(base + b * CH, CH)])
    plsc.subcore_barrier()

    pltpu.sync_copy(dst_hbm.at[w], dstv)

    def chunk(j, carry):
        pltpu.sync_copy(onesv, acc.at[dstv.at[j]], add=True)
        return carry

    lax.fori_loop(0, NCHUNK, chunk, 0)
    plsc.subcore_barrier()

    for b in range(RPT // CH):
        r0 = base + b * CH
        pltpu.sync_copy(acc.at[pl.ds(r0, CH)], stg)
        pltpu.sync_copy(stg, out_hbm.at[c, pl.ds(r0, CH)])


@functools.partial(
    pl.kernel,
    out_type=jax.ShapeDtypeStruct((NC, N_PAD, D_FEAT), jnp.float32),
    mesh=_mesh,
    scratch_types=[
        pltpu.VMEM((GC, CH), jnp.int32),
        pltpu.VMEM((NCHUNK, CH), jnp.int32),
        pltpu.VMEM((CH, D_FEAT), jnp.float32),
        pltpu.VMEM((CH, D_FEAT), jnp.float32),
        pltpu.VMEM_SHARED((N_PAD, D_FEAT), jnp.float32),
        pltpu.SemaphoreType.DMA,
        pltpu.SemaphoreType.DMA,
        pltpu.SemaphoreType.DMA,
        pltpu.SemaphoreType.DMA,
    ],
)
def _hop_kernel(
    g_hbm, src_hbm, dst_hbm, out_hbm, srcg, dstv, rows0, rows1, acc,
    sem0, sem1, ssem0, ssem1
):
    c = lax.axis_index("c")
    s = lax.axis_index("s")
    w = c * NS + s

    _zero_vmem(rows0, CH, D_FEAT)
    base = s * RPT
    for b in range(RPT // CH):
        pltpu.sync_copy(rows0, acc.at[pl.ds(base + b * CH, CH)])
    plsc.subcore_barrier()

    pltpu.sync_copy(dst_hbm.at[w], dstv)

    bufs = (rows0, rows1)
    sems = (sem0, sem1)
    ssems = (ssem0, ssem1)

    def group(g, carry):
        pltpu.sync_copy(src_hbm.at[w, g], srcg)
        pltpu.async_copy(g_hbm.at[srcg.at[0]], rows0, sem0)
        pltpu.async_copy(g_hbm.at[srcg.at[1]], rows1, sem1)
        scat = [None, None]
        for k in range(GC):
            b = k % 2
            buf, sem = bufs[b], sems[b]
            pltpu.make_async_copy(g_hbm.at[srcg.at[k]], buf, sem).wait()
            scat[b] = pltpu.async_copy(buf, acc.at[dstv.at[g * GC + k]], ssems[b], add=True)
            if k + 2 < GC:
                scat[b].wait()
                pltpu.async_copy(g_hbm.at[srcg.at[k + 2]], buf, sem)
        scat[0].wait()
        scat[1].wait()
        return carry

    lax.fori_loop(0, NG, group, 0)
    plsc.subcore_barrier()

    for b in range(RPT // CH):
        r0 = base + b * CH
        pltpu.sync_copy(acc.at[pl.ds(r0, CH)], rows0)
        pltpu.sync_copy(rows0, out_hbm.at[c, pl.ds(r0, CH)])


_ROWS_BLK = 1000
_GRID = N_NODES // _ROWS_BLK


def _prep_body(f_ref, degp_ref, g0_ref, nrm_ref):
    deg = jnp.maximum(degp_ref[0][:, 0:1] + degp_ref[1][:, 0:1], 1.0)
    rn = lax.rsqrt(deg)
    inv = 1.0 / deg
    g0_ref[...] = f_ref[...] * rn
    nrm_ref[...] = jnp.concatenate([rn, inv, rn, inv, rn, inv, rn, inv], axis=1)


def _mid_body(p_ref, nrm_ref, g1_ref):
    g1_ref[...] = (p_ref[0] + p_ref[1]) * nrm_ref[:, 1:2]


def _head_body(q_ref, nrm_ref, wsg_ref, bsg_ref, wlin_ref, blin_ref, o_ref):
    h = (q_ref[0] + q_ref[1]) * nrm_ref[:, 0:1]
    z = jnp.dot(h, wsg_ref[...], preferred_element_type=jnp.float32) + bsg_ref[...]
    o_ref[...] = (
        jnp.dot(jnp.maximum(z, 0.0), wlin_ref[...], preferred_element_type=jnp.float32)
        + blin_ref[...]
    )


_degp_spec = pl.BlockSpec((NC, _ROWS_BLK, DEGW), lambda i: (0, i, 0))
_full_spec = pl.BlockSpec((_ROWS_BLK, D_FEAT), lambda i: (i, 0))
_pair_spec = pl.BlockSpec((NC, _ROWS_BLK, D_FEAT), lambda i: (0, i, 0))
_nrm_spec = pl.BlockSpec((_ROWS_BLK, 8), lambda i: (i, 0))

_prep = pl.pallas_call(
    _prep_body,
    grid=(_GRID,),
    in_specs=[_full_spec, _degp_spec],
    out_specs=(_full_spec, _nrm_spec),
    out_shape=(
        jax.ShapeDtypeStruct((N_NODES, D_FEAT), jnp.float32),
        jax.ShapeDtypeStruct((N_NODES, 8), jnp.float32),
    ),
)

_mid = pl.pallas_call(
    _mid_body,
    grid=(_GRID,),
    in_specs=[_pair_spec, _nrm_spec],
    out_specs=_full_spec,
    out_shape=jax.ShapeDtypeStruct((N_NODES, D_FEAT), jnp.float32),
)

_head = pl.pallas_call(
    _head_body,
    grid=(_GRID,),
    in_specs=[
        _pair_spec,
        _nrm_spec,
        pl.BlockSpec((D_FEAT, D_FEAT), lambda i: (0, 0)),
        pl.BlockSpec((1, D_FEAT), lambda i: (0, 0)),
        pl.BlockSpec((D_FEAT, N_CLASSES), lambda i: (0, 0)),
        pl.BlockSpec((1, N_CLASSES), lambda i: (0, 0)),
    ],
    out_specs=pl.BlockSpec((_ROWS_BLK, N_CLASSES), lambda i: (i, 0)),
    out_shape=jax.ShapeDtypeStruct((N_NODES, N_CLASSES), jnp.float32),
)


def kernel(features, edge_index, W_sg, b_sg, W_lin, b_lin):
    n_extra = E_PAD - N_EDGES
    pad_ids = jnp.arange(n_extra, dtype=jnp.int32)
    pad_src = pad_ids % N_NODES
    pad_dst = N_NODES + pad_ids % (N_PAD - N_NODES)
    src = jnp.concatenate([edge_index[0], pad_src]).reshape(NW, NG, GC, CH)
    dst = jnp.concatenate([edge_index[1], pad_dst]).reshape(NW, NCHUNK, CH)
    degp = _deg_kernel(dst)
    g0, nrm = _prep(features, degp)
    p1 = _hop_kernel(g0, src, dst)
    g1 = _mid(p1, nrm)
    p2 = _hop_kernel(g1, src, dst)
    return _head(
        p2, nrm, W_sg, b_sg.reshape(1, D_FEAT), W_lin, b_lin.reshape(1, N_CLASSES)
    )


# speedup vs baseline: 9.1434x; 1.0472x over previous
import functools

import jax
import jax.numpy as jnp
from jax import lax
from jax.experimental import pallas as pl
from jax.experimental.pallas import tpu as pltpu
from jax.experimental.pallas import tpu_sc as plsc

N_NODES = 10000
N_EDGES = 320000
D_FEAT = 128
N_CLASSES = 40

NC = 2
NS = 16
NW = NC * NS
CH = 128
GC = 16
E_PAD = 327680
EPW = E_PAD // NW
NCHUNK = EPW // CH
NG = NCHUNK // GC
N_PAD = 10240
RPT = N_PAD // NS
DEGW = 128

_mesh = plsc.VectorSubcoreMesh(core_axis_name="c", subcore_axis_name="s")


def _zero_vmem(ref, nrows, width):
    z = jnp.zeros((16,), jnp.float32)

    def body(i, carry):
        for k in range(width // 16):
            ref[i, pl.ds(k * 16, 16)] = z
        return carry

    lax.fori_loop(0, nrows, body, 0)


@functools.partial(
    pl.kernel,
    out_type=jax.ShapeDtypeStruct((NC, N_PAD, DEGW), jnp.float32),
    mesh=_mesh,
    scratch_types=[
        pltpu.VMEM((NCHUNK, CH), jnp.int32),
        pltpu.VMEM((CH, DEGW), jnp.float32),
        pltpu.VMEM((CH, DEGW), jnp.float32),
        pltpu.VMEM_SHARED((N_PAD, DEGW), jnp.float32),
    ],
)
def _deg_kernel(dst_hbm, out_hbm, dstv, onesv, stg, acc):
    c = lax.axis_index("c")
    s = lax.axis_index("s")
    w = c * NS + s
    one = jnp.ones((16,), jnp.float32)

    def init_ones(i, carry):
        for k in range(DEGW // 16):
            onesv[i, pl.ds(k * 16, 16)] = one
        return carry

    lax.fori_loop(0, CH, init_ones, 0)
    _zero_vmem(stg, CH, DEGW)
    base = s * RPT
    for b in range(RPT // CH):
        pltpu.sync_copy(stg, acc.at[pl.ds(base + b * CH, CH)])
    plsc.subcore_barrier()

    pltpu.sync_copy(dst_hbm.at[w], dstv)

    def chunk(j, carry):
        pltpu.sync_copy(onesv, acc.at[dstv.at[j]], add=True)
        return carry

    lax.fori_loop(0, NCHUNK, chunk, 0)
    plsc.subcore_barrier()

    for b in range(RPT // CH):
        r0 = base + b * CH
        pltpu.sync_copy(acc.at[pl.ds(r0, CH)], stg)
        pltpu.sync_copy(stg, out_hbm.at[c, pl.ds(r0, CH)])


@functools.partial(
    pl.kernel,
    out_type=jax.ShapeDtypeStruct((NC, N_PAD, D_FEAT), jnp.float32),
    mesh=_mesh,
    scratch_types=[
        pltpu.VMEM((GC, CH), jnp.int32),
        pltpu.VMEM((NCHUNK, CH), jnp.int32),
        pltpu.VMEM((CH, D_FEAT), jnp.float32),
        pltpu.VMEM((CH, D_FEAT), jnp.float32),
        pltpu.VMEM_SHARED((N_PAD, D_FEAT), jnp.float32),
        pltpu.SemaphoreType.DMA,
        pltpu.SemaphoreType.DMA,
        pltpu.SemaphoreType.DMA,
        pltpu.SemaphoreType.DMA,
    ],
)
def _hop_kernel(
    g_hbm, src_hbm, dst_hbm, out_hbm, srcg, dstv, rows0, rows1, acc,
    sem0, sem1, ssem0, ssem1
):
    c = lax.axis_index("c")
    s = lax.axis_index("s")
    w = c * NS + s

    _zero_vmem(rows0, CH, D_FEAT)
    base = s * RPT
    for b in range(RPT // CH):
        pltpu.sync_copy(rows0, acc.at[pl.ds(base + b * CH, CH)])
    plsc.subcore_barrier()

    pltpu.sync_copy(dst_hbm.at[w], dstv)

    bufs = (rows0, rows1)
    sems = (sem0, sem1)
    ssems = (ssem0, ssem1)

    def group(g, carry):
        pltpu.sync_copy(src_hbm.at[w, g], srcg)
        pltpu.async_copy(g_hbm.at[srcg.at[0]], rows0, sem0)
        pltpu.async_copy(g_hbm.at[srcg.at[1]], rows1, sem1)
        scat = [None, None]
        for k in range(GC):
            b = k % 2
            buf, sem = bufs[b], sems[b]
            pltpu.make_async_copy(g_hbm.at[srcg.at[k]], buf, sem).wait()
            scat[b] = pltpu.async_copy(buf, acc.at[dstv.at[g * GC + k]], ssems[b], add=True)
            if k + 2 < GC:
                scat[b].wait()
                pltpu.async_copy(g_hbm.at[srcg.at[k + 2]], buf, sem)
        scat[0].wait()
        scat[1].wait()
        return carry

    lax.fori_loop(0, NG, group, 0)
    plsc.subcore_barrier()

    for b in range(RPT // CH):
        r0 = base + b * CH
        pltpu.sync_copy(acc.at[pl.ds(r0, CH)], rows0)
        pltpu.sync_copy(rows0, out_hbm.at[c, pl.ds(r0, CH)])


_ROWS_BLK = 1000
_GRID = N_NODES // _ROWS_BLK


def _prep_body(f_ref, degp_ref, g0_ref, nrm_ref):
    deg = jnp.maximum(degp_ref[0][:, 0:1] + degp_ref[1][:, 0:1], 1.0)
    rn = lax.rsqrt(deg)
    inv = 1.0 / deg
    g0_ref[...] = f_ref[...] * rn
    nrm_ref[...] = jnp.concatenate([rn, inv, rn, inv, rn, inv, rn, inv], axis=1)


def _mid_body(p_ref, nrm_ref, g1_ref):
    g1_ref[...] = (p_ref[0] + p_ref[1]) * nrm_ref[:, 1:2]


def _head_body(q_ref, nrm_ref, wsg_ref, bsg_ref, wlin_ref, blin_ref, o_ref):
    h = (q_ref[0] + q_ref[1]) * nrm_ref[:, 0:1]
    z = jnp.dot(h, wsg_ref[...], preferred_element_type=jnp.float32) + bsg_ref[...]
    o_ref[...] = (
        jnp.dot(jnp.maximum(z, 0.0), wlin_ref[...], preferred_element_type=jnp.float32)
        + blin_ref[...]
    )


_degp_spec = pl.BlockSpec((NC, _ROWS_BLK, DEGW), lambda i: (0, i, 0))
_full_spec = pl.BlockSpec((_ROWS_BLK, D_FEAT), lambda i: (i, 0))
_pair_spec = pl.BlockSpec((NC, _ROWS_BLK, D_FEAT), lambda i: (0, i, 0))
_nrm_spec = pl.BlockSpec((_ROWS_BLK, 8), lambda i: (i, 0))

_prep = pl.pallas_call(
    _prep_body,
    grid=(_GRID,),
    in_specs=[_full_spec, _degp_spec],
    out_specs=(_full_spec, _nrm_spec),
    out_shape=(
        jax.ShapeDtypeStruct((N_NODES, D_FEAT), jnp.float32),
        jax.ShapeDtypeStruct((N_NODES, 8), jnp.float32),
    ),
)

_mid = pl.pallas_call(
    _mid_body,
    grid=(_GRID,),
    in_specs=[_pair_spec, _nrm_spec],
    out_specs=_full_spec,
    out_shape=jax.ShapeDtypeStruct((N_NODES, D_FEAT), jnp.float32),
)

_head = pl.pallas_call(
    _head_body,
    grid=(_GRID,),
    in_specs=[
        _pair_spec,
        _nrm_spec,
        pl.BlockSpec((D_FEAT, D_FEAT), lambda i: (0, 0)),
        pl.BlockSpec((1, D_FEAT), lambda i: (0, 0)),
        pl.BlockSpec((D_FEAT, N_CLASSES), lambda i: (0, 0)),
        pl.BlockSpec((1, N_CLASSES), lambda i: (0, 0)),
    ],
    out_specs=pl.BlockSpec((_ROWS_BLK, N_CLASSES), lambda i: (i, 0)),
    out_shape=jax.ShapeDtypeStruct((N_NODES, N_CLASSES), jnp.float32),
)


def kernel(features, edge_index, W_sg, b_sg, W_lin, b_lin):
    n_extra = E_PAD - N_EDGES
    pad_ids = jnp.arange(n_extra, dtype=jnp.int32)
    pad_src = pad_ids % N_NODES
    pad_dst = N_NODES + pad_ids % (N_PAD - N_NODES)
    src = jnp.concatenate([edge_index[0], pad_src]).reshape(NW, NG, GC, CH)
    dst = jnp.concatenate([edge_index[1], pad_dst]).reshape(NW, NCHUNK, CH)
    degp = _deg_kernel(dst)
    g0, nrm = _prep(features, degp)
    p1 = _hop_kernel(g0, src, dst)
    g1 = _mid(p1, nrm)
    p2 = _hop_kernel(g1, src, dst)
    return _head(
        p2, nrm, W_sg, b_sg.reshape(1, D_FEAT), W_lin, b_lin.reshape(1, N_CLASSES)
    )


# speedup vs baseline: 9.2422x; 1.0108x over previous
import functools

import jax
import jax.numpy as jnp
from jax import lax
from jax.experimental import pallas as pl
from jax.experimental.pallas import tpu as pltpu
from jax.experimental.pallas import tpu_sc as plsc

N_NODES = 10000
N_EDGES = 320000
D_FEAT = 128
N_CLASSES = 40

NC = 2
NS = 16
NW = NC * NS
CH = 128
GC = 20
E_PAD = 327680
EPW = E_PAD // NW
NCHUNK = EPW // CH
NG = NCHUNK // GC
N_PAD = 10240
RPT = N_PAD // NS
DEGW = 128

_mesh = plsc.VectorSubcoreMesh(core_axis_name="c", subcore_axis_name="s")


def _zero_vmem(ref, nrows, width):
    z = jnp.zeros((16,), jnp.float32)

    def body(i, carry):
        for k in range(width // 16):
            ref[i, pl.ds(k * 16, 16)] = z
        return carry

    lax.fori_loop(0, nrows, body, 0)


@functools.partial(
    pl.kernel,
    out_type=jax.ShapeDtypeStruct((NC, N_PAD, DEGW), jnp.float32),
    mesh=_mesh,
    scratch_types=[
        pltpu.VMEM((NCHUNK, CH), jnp.int32),
        pltpu.VMEM((CH, DEGW), jnp.float32),
        pltpu.VMEM((CH, DEGW), jnp.float32),
        pltpu.VMEM_SHARED((N_PAD, DEGW), jnp.float32),
    ],
)
def _deg_kernel(dst_hbm, out_hbm, dstv, onesv, stg, acc):
    c = lax.axis_index("c")
    s = lax.axis_index("s")
    w = c * NS + s
    one = jnp.ones((16,), jnp.float32)

    def init_ones(i, carry):
        for k in range(DEGW // 16):
            onesv[i, pl.ds(k * 16, 16)] = one
        return carry

    lax.fori_loop(0, CH, init_ones, 0)
    _zero_vmem(stg, CH, DEGW)
    base = s * RPT
    for b in range(RPT // CH):
        pltpu.sync_copy(stg, acc.at[pl.ds(base + b * CH, CH)])
    plsc.subcore_barrier()

    pltpu.sync_copy(dst_hbm.at[w], dstv)

    def chunk(j, carry):
        pltpu.sync_copy(onesv, acc.at[dstv.at[j]], add=True)
        return carry

    lax.fori_loop(0, NCHUNK, chunk, 0)
    plsc.subcore_barrier()

    for b in range(RPT // CH):
        r0 = base + b * CH
        pltpu.sync_copy(acc.at[pl.ds(r0, CH)], stg)
        pltpu.sync_copy(stg, out_hbm.at[c, pl.ds(r0, CH)])


@functools.partial(
    pl.kernel,
    out_type=jax.ShapeDtypeStruct((NC, N_PAD, D_FEAT), jnp.float32),
    mesh=_mesh,
    scratch_types=[
        pltpu.VMEM((GC, CH), jnp.int32),
        pltpu.VMEM((NCHUNK, CH), jnp.int32),
        pltpu.VMEM((CH, D_FEAT), jnp.float32),
        pltpu.VMEM((CH, D_FEAT), jnp.float32),
        pltpu.VMEM_SHARED((N_PAD, D_FEAT), jnp.float32),
        pltpu.SemaphoreType.DMA,
        pltpu.SemaphoreType.DMA,
        pltpu.SemaphoreType.DMA,
        pltpu.SemaphoreType.DMA,
    ],
)
def _hop_kernel(
    g_hbm, src_hbm, dst_hbm, out_hbm, srcg, dstv, rows0, rows1, acc,
    sem0, sem1, ssem0, ssem1
):
    c = lax.axis_index("c")
    s = lax.axis_index("s")
    w = c * NS + s

    _zero_vmem(rows0, CH, D_FEAT)
    base = s * RPT
    for b in range(RPT // CH):
        pltpu.sync_copy(rows0, acc.at[pl.ds(base + b * CH, CH)])
    plsc.subcore_barrier()

    pltpu.sync_copy(dst_hbm.at[w], dstv)

    bufs = (rows0, rows1)
    sems = (sem0, sem1)
    ssems = (ssem0, ssem1)

    def group(g, carry):
        pltpu.sync_copy(src_hbm.at[w, g], srcg)
        pltpu.async_copy(g_hbm.at[srcg.at[0]], rows0, sem0)
        pltpu.async_copy(g_hbm.at[srcg.at[1]], rows1, sem1)
        scat = [None, None]
        for k in range(GC):
            b = k % 2
            buf, sem = bufs[b], sems[b]
            pltpu.make_async_copy(g_hbm.at[srcg.at[k]], buf, sem).wait()
            scat[b] = pltpu.async_copy(buf, acc.at[dstv.at[g * GC + k]], ssems[b], add=True)
            if k + 2 < GC:
                scat[b].wait()
                pltpu.async_copy(g_hbm.at[srcg.at[k + 2]], buf, sem)
        scat[0].wait()
        scat[1].wait()
        return carry

    lax.fori_loop(0, NG, group, 0)
    plsc.subcore_barrier()

    for b in range(RPT // CH):
        r0 = base + b * CH
        pltpu.sync_copy(acc.at[pl.ds(r0, CH)], rows0)
        pltpu.sync_copy(rows0, out_hbm.at[c, pl.ds(r0, CH)])


_ROWS_BLK = 1000
_GRID = N_NODES // _ROWS_BLK


def _prep_body(f_ref, degp_ref, g0_ref, nrm_ref):
    deg = jnp.maximum(degp_ref[0][:, 0:1] + degp_ref[1][:, 0:1], 1.0)
    rn = lax.rsqrt(deg)
    inv = 1.0 / deg
    g0_ref[...] = f_ref[...] * rn
    nrm_ref[...] = jnp.concatenate([rn, inv, rn, inv, rn, inv, rn, inv], axis=1)


def _mid_body(p_ref, nrm_ref, g1_ref):
    g1_ref[...] = (p_ref[0] + p_ref[1]) * nrm_ref[:, 1:2]


def _head_body(q_ref, nrm_ref, wsg_ref, bsg_ref, wlin_ref, blin_ref, o_ref):
    h = (q_ref[0] + q_ref[1]) * nrm_ref[:, 0:1]
    z = jnp.dot(h, wsg_ref[...], preferred_element_type=jnp.float32) + bsg_ref[...]
    o_ref[...] = (
        jnp.dot(jnp.maximum(z, 0.0), wlin_ref[...], preferred_element_type=jnp.float32)
        + blin_ref[...]
    )


_degp_spec = pl.BlockSpec((NC, _ROWS_BLK, DEGW), lambda i: (0, i, 0))
_full_spec = pl.BlockSpec((_ROWS_BLK, D_FEAT), lambda i: (i, 0))
_pair_spec = pl.BlockSpec((NC, _ROWS_BLK, D_FEAT), lambda i: (0, i, 0))
_nrm_spec = pl.BlockSpec((_ROWS_BLK, 8), lambda i: (i, 0))

_prep = pl.pallas_call(
    _prep_body,
    grid=(_GRID,),
    in_specs=[_full_spec, _degp_spec],
    out_specs=(_full_spec, _nrm_spec),
    out_shape=(
        jax.ShapeDtypeStruct((N_NODES, D_FEAT), jnp.float32),
        jax.ShapeDtypeStruct((N_NODES, 8), jnp.float32),
    ),
)

_mid = pl.pallas_call(
    _mid_body,
    grid=(_GRID,),
    in_specs=[_pair_spec, _nrm_spec],
    out_specs=_full_spec,
    out_shape=jax.ShapeDtypeStruct((N_NODES, D_FEAT), jnp.float32),
)

_head = pl.pallas_call(
    _head_body,
    grid=(_GRID,),
    in_specs=[
        _pair_spec,
        _nrm_spec,
        pl.BlockSpec((D_FEAT, D_FEAT), lambda i: (0, 0)),
        pl.BlockSpec((1, D_FEAT), lambda i: (0, 0)),
        pl.BlockSpec((D_FEAT, N_CLASSES), lambda i: (0, 0)),
        pl.BlockSpec((1, N_CLASSES), lambda i: (0, 0)),
    ],
    out_specs=pl.BlockSpec((_ROWS_BLK, N_CLASSES), lambda i: (i, 0)),
    out_shape=jax.ShapeDtypeStruct((N_NODES, N_CLASSES), jnp.float32),
)


def kernel(features, edge_index, W_sg, b_sg, W_lin, b_lin):
    n_extra = E_PAD - N_EDGES
    pad_ids = jnp.arange(n_extra, dtype=jnp.int32)
    pad_src = pad_ids % N_NODES
    pad_dst = N_NODES + pad_ids % (N_PAD - N_NODES)
    src = jnp.concatenate([edge_index[0], pad_src]).reshape(NW, NG, GC, CH)
    dst = jnp.concatenate([edge_index[1], pad_dst]).reshape(NW, NCHUNK, CH)
    degp = _deg_kernel(dst)
    g0, nrm = _prep(features, degp)
    p1 = _hop_kernel(g0, src, dst)
    g1 = _mid(p1, nrm)
    p2 = _hop_kernel(g1, src, dst)
    return _head(
        p2, nrm, W_sg, b_sg.reshape(1, D_FEAT), W_lin, b_lin.reshape(1, N_CLASSES)
    )
